# baseline (speedup 1.0000x reference)
import functools

import jax
import jax.numpy as jnp
from jax import lax
from jax.experimental import pallas as pl
from jax.experimental.pallas import tpu as pltpu
from jax.experimental.pallas import tpu_sc as plsc

HIDDEN = 32
OUT = 16
GROUP = 128 // HIDDEN


def _sc_gather_lines(table4, line_idx):
    info = plsc.get_sparse_core_info()
    nc, ns = info.num_cores, info.num_subcores
    nw = nc * ns
    b = line_idx.shape[0]
    assert b % (8 * nw) == 0
    b_per_w = b // nw
    mesh = plsc.VectorSubcoreMesh(core_axis_name="c", subcore_axis_name="s")

    @functools.partial(
        pl.kernel,
        mesh=mesh,
        out_type=jax.ShapeDtypeStruct((b, 128), jnp.float32),
        scratch_types=[
            pltpu.VMEM((b_per_w,), jnp.int32),
            pltpu.VMEM((b_per_w, 128), jnp.float32),
            pltpu.SemaphoreType.DMA,
        ],
    )
    def gather_kernel(table_hbm, idx_hbm, out_hbm, idx_v, rows_v, sem):
        wid = lax.axis_index("s") * nc + lax.axis_index("c")
        base = wid * b_per_w
        pltpu.sync_copy(idx_hbm.at[pl.ds(base, b_per_w)], idx_v)
        pltpu.async_copy(table_hbm.at[idx_v], rows_v, sem).wait()
        pltpu.sync_copy(rows_v, out_hbm.at[pl.ds(base, b_per_w)])

    return gather_kernel(table4, line_idx)


def _mlp_body(g4_ref, m_ref, w_ref, b_ref, o_ref):
    g4 = g4_ref[...]
    m = m_ref[...]
    h = g4[:, 0:HIDDEN]
    for k in range(1, GROUP):
        h = jnp.where(m == k, g4[:, k * HIDDEN:(k + 1) * HIDDEN], h)
    h = jnp.maximum(h, 0.0)
    o_ref[...] = (
        lax.dot_general(
            h, w_ref[...], (((1,), (1,)), ((), ())),
            preferred_element_type=jnp.float32,
        )
        + b_ref[...]
    )


def kernel(x, emb, W2, b2):
    b = x.shape[0]
    v = emb.shape[0]
    idx = x.reshape(b).astype(jnp.int32)
    table4 = emb.reshape(v // GROUP, GROUP * HIDDEN)
    g4 = _sc_gather_lines(table4, idx // GROUP)
    m = (idx % GROUP).reshape(b, 1)
    y = pl.pallas_call(
        _mlp_body,
        out_shape=jax.ShapeDtypeStruct((b, OUT), jnp.float32),
    )(g4, m, W2, b2.reshape(1, OUT))
    return y
